# baseline (speedup 1.0000x reference)
import functools

import jax
import jax.numpy as jnp
from jax import lax
from jax.experimental import pallas as pl
from jax.experimental.pallas import tpu as pltpu
from jax.experimental.pallas import tpu_sc as plsc

_NUM_CORES = 2
_NUM_SUBCORES = 16
_NUM_WORKERS = _NUM_CORES * _NUM_SUBCORES
_CHUNK = 32
_RING = 3


def _copy_body(rows_per_worker, n_chunks, w_hbm, out_hbm,
               b0, b1, b2, si0, si1, si2, so0, so1, so2):
    wid = lax.axis_index("s") * _NUM_CORES + lax.axis_index("c")
    base = wid * rows_per_worker
    bufs = (b0, b1, b2)
    sin = (si0, si1, si2)
    sout = (so0, so1, so2)

    def load(i):
        return pltpu.async_copy(
            w_hbm.at[pl.ds(base + i * _CHUNK, _CHUNK)],
            bufs[i % _RING], sin[i % _RING])

    def store(i):
        return pltpu.async_copy(
            bufs[i % _RING],
            out_hbm.at[pl.ds(base + i * _CHUNK, _CHUNK)], sout[i % _RING])

    loads = {}
    for j in range(min(_RING, n_chunks)):
        loads[j] = load(j)
    stores = {}
    for i in range(n_chunks):
        loads[i].wait()
        stores[i] = store(i)
        j = i + _RING
        if j < n_chunks:
            stores[i].wait()
            loads[j] = load(j)
    for i in range(max(0, n_chunks - _RING), n_chunks):
        stores[i].wait()


def kernel(x, W):
    seq_len = x.shape[1]
    d_model = W.shape[1]
    assert seq_len % (_NUM_WORKERS * _CHUNK) == 0
    rows_per_worker = seq_len // _NUM_WORKERS
    n_chunks = rows_per_worker // _CHUNK

    mesh = plsc.VectorSubcoreMesh(core_axis_name="c", subcore_axis_name="s")
    body = functools.partial(_copy_body, rows_per_worker, n_chunks)
    f = pl.kernel(
        body,
        mesh=mesh,
        out_type=jax.ShapeDtypeStruct((seq_len, d_model), W.dtype),
        scratch_types=(
            [pltpu.VMEM((_CHUNK, d_model), W.dtype) for _ in range(_RING)]
            + [pltpu.SemaphoreType.DMA for _ in range(2 * _RING)]
        ),
    )
    out = f(W)
    return out[None]
